# baseline (speedup 1.0000x reference)
import jax
import jax.numpy as jnp
from jax.experimental import pallas as pl
from jax.experimental.pallas import tpu as pltpu

_OFFSETS = [(-1, -1), (-1, 0), (-1, 1),
            (0, -1), (0, 0), (0, 1),
            (1, -1), (1, 0), (1, 1)]


def _patches(x):
    b, h, w, c = x.shape
    zrow = jnp.zeros((b, 1, w, c), x.dtype)
    xp = jnp.concatenate([zrow, x, zrow], axis=1)
    zcol = jnp.zeros((b, h + 2, 1, c), x.dtype)
    xp = jnp.concatenate([zcol, xp, zcol], axis=2)
    slabs = [xp[:, 1 + dh:1 + dh + h, 1 + dw:1 + dw + w, :]
             for dh, dw in _OFFSETS]
    cols = jnp.concatenate(slabs, axis=-1)
    return cols.reshape(b * h * w, 9 * c)


def _pixel_shuffle(y, b, h, w, c):
    quads = [y[:, p * c:(p + 1) * c].reshape(b, h, w, 1, c) for p in range(4)]
    row0 = jnp.concatenate([quads[0], quads[1]], axis=3).reshape(b, h, 1, 2 * w, c)
    row1 = jnp.concatenate([quads[2], quads[3]], axis=3).reshape(b, h, 1, 2 * w, c)
    return jnp.concatenate([row0, row1], axis=2).reshape(b, 2 * h, 2 * w, c)


def _col_stats(acc):
    s = jnp.sum(acc, axis=0, keepdims=True)
    q = jnp.sum(acc * acc, axis=0, keepdims=True)
    return jnp.concatenate([s[:, None, :], q[:, None, :]], axis=1)


def _finish_bn(acc, sums, g, t, m_full):
    n = acc.shape[1]
    c = n // 4
    col_sum = sums[0, 0:1, :] + sums[1, 0:1, :]
    col_sq = sums[0, 1:2, :] + sums[1, 1:2, :]
    ch_sum = (col_sum[:, 0:c] + col_sum[:, c:2 * c]
              + col_sum[:, 2 * c:3 * c] + col_sum[:, 3 * c:4 * c])
    ch_sq = (col_sq[:, 0:c] + col_sq[:, c:2 * c]
             + col_sq[:, 2 * c:3 * c] + col_sq[:, 3 * c:4 * c])
    cnt = jnp.float32(m_full * 4)
    mean = ch_sum / cnt
    var = ch_sq / cnt - mean * mean
    inv = jax.lax.rsqrt(var + 1e-5)
    scale = g[:, 0:c] * inv
    shift = t[:, 0:c] - mean * scale
    scale = jnp.concatenate([scale] * 4, axis=1)
    shift = jnp.concatenate([shift] * 4, axis=1)
    acc = acc * scale + shift
    return jnp.where(acc >= 0, acc, 0.05 * acc)


def _fc_d1_kernel(x_ref, w0_ref, b0_ref, w1_ref, acc_ref, sums_ref):
    bh = x_ref.shape[0]
    y0 = jnp.dot(x_ref[...], w0_ref[...],
                 preferred_element_type=jnp.float32) + b0_ref[...]
    y0 = y0.astype(jnp.bfloat16)
    c0 = 512
    cell = [y0[:, q * c0:(q + 1) * c0].reshape(bh, 1, 1, c0) for q in range(4)]
    top = jnp.concatenate([cell[0], cell[1]], axis=2)
    bot = jnp.concatenate([cell[2], cell[3]], axis=2)
    x0 = jnp.concatenate([top, bot], axis=1)
    acc = jnp.dot(_patches(x0), w1_ref[...], preferred_element_type=jnp.float32)
    sums_ref[...] = _col_stats(acc)
    acc_ref[...] = acc.astype(jnp.bfloat16)


def _mk_bn_step(h, w, cin, m_full):
    def body(acc_ref, sums_ref, g_ref, t_ref, w_ref, oacc_ref, osums_ref):
        bh = acc_ref.shape[0] // (h * w)
        acc = _finish_bn(acc_ref[...].astype(jnp.float32), sums_ref[...],
                         g_ref[...], t_ref[...], m_full)
        x = _pixel_shuffle(acc.astype(jnp.bfloat16), bh, h, w, cin)
        nacc = jnp.dot(_patches(x), w_ref[...],
                       preferred_element_type=jnp.float32)
        osums_ref[...] = _col_stats(nacc)
        oacc_ref[...] = nacc.astype(jnp.bfloat16)
    return body


def _mk_tail(m_full):
  def _tail_kernel(acc_ref, sums_ref, g_ref, t_ref, w_ref, b_ref, o_ref):
    m = acc_ref.shape[0]
    bh = m // 64
    acc = _finish_bn(acc_ref[...].astype(jnp.float32), sums_ref[...],
                     g_ref[...], t_ref[...], m_full)
    y = acc.astype(jnp.bfloat16)
    z = jnp.zeros((m, 64), jnp.bfloat16)
    yp = jnp.concatenate([y[:, 0:64], z, y[:, 64:128], z,
                          y[:, 128:192], z, y[:, 192:256], z], axis=1)
    x3 = _pixel_shuffle(yp, bh, 8, 8, 128)
    zrow = jnp.zeros((bh, 1, 16, 128), jnp.bfloat16)
    xp = jnp.concatenate([zrow, x3, zrow], axis=1)
    zcol = jnp.zeros((bh, 18, 1, 128), jnp.bfloat16)
    xp = jnp.concatenate([zcol, xp, zcol], axis=2)
    out = jnp.zeros((bh * 256, 12), jnp.float32)
    for t, (dh, dw) in enumerate(_OFFSETS):
        slab = xp[:, 1 + dh:1 + dh + 16, 1 + dw:1 + dw + 16, :]
        out = out + jnp.dot(slab.reshape(bh * 256, 128),
                            w_ref[t * 128:(t + 1) * 128, :],
                            preferred_element_type=jnp.float32)
    o_ref[...] = jnp.tanh(out + b_ref[...])
  return _tail_kernel


def kernel(z, tags, wmat_0, bias_0, gamma_0, beta_0,
           wmat_1, bias_1, gamma_1, beta_1,
           wmat_2, bias_2, gamma_2, beta_2,
           wmat_3, bias_3, gamma_3, beta_3,
           wmat_4, bias_4, gamma_4, beta_4):
    bsz = z.shape[0]
    x = jnp.concatenate([z, tags], axis=1).astype(jnp.bfloat16)

    par = pltpu.CompilerParams(dimension_semantics=("parallel",))
    rep = lambda shape: pl.BlockSpec(shape, lambda g: tuple(0 for _ in shape))
    half = lambda shape: pl.BlockSpec(shape, lambda g: (g,) + tuple(
        0 for _ in shape[1:]))

    def stats_shape(n):
        return jax.ShapeDtypeStruct((2, 2, n), jnp.float32)

    acc1, sums1 = pl.pallas_call(
        _fc_d1_kernel,
        grid=(2,),
        out_shape=(jax.ShapeDtypeStruct((bsz * 4, 1024), jnp.bfloat16),
                   stats_shape(1024)),
        in_specs=[half((bsz // 2, x.shape[1])),
                  rep(wmat_0.shape), rep(bias_0.shape), rep(wmat_1.shape)],
        out_specs=(half((bsz * 2, 1024)), half((1, 2, 1024))),
        compiler_params=par,
    )(x, wmat_0, bias_0, wmat_1)

    acc2, sums2 = pl.pallas_call(
        _mk_bn_step(2, 2, 256, bsz * 4),
        grid=(2,),
        out_shape=(jax.ShapeDtypeStruct((bsz * 16, 512), jnp.bfloat16),
                   stats_shape(512)),
        in_specs=[half((bsz * 2, 1024)), rep((2, 2, 1024)),
                  rep(gamma_1.shape), rep(beta_1.shape), rep(wmat_2.shape)],
        out_specs=(half((bsz * 8, 512)), half((1, 2, 512))),
        compiler_params=par,
    )(acc1, sums1, gamma_1, beta_1, wmat_2)

    acc3, sums3 = pl.pallas_call(
        _mk_bn_step(4, 4, 128, bsz * 16),
        grid=(2,),
        out_shape=(jax.ShapeDtypeStruct((bsz * 64, 256), jnp.bfloat16),
                   stats_shape(256)),
        in_specs=[half((bsz * 8, 512)), rep((2, 2, 512)),
                  rep(gamma_2.shape), rep(beta_2.shape), rep(wmat_3.shape)],
        out_specs=(half((bsz * 32, 256)), half((1, 2, 256))),
        compiler_params=par,
    )(acc2, sums2, gamma_2, beta_2, wmat_3)

    w4r = wmat_4.reshape(9, 64, 12)
    w4p = jnp.concatenate([w4r, jnp.zeros((9, 64, 12), wmat_4.dtype)],
                          axis=1).reshape(1152, 12)
    y4 = pl.pallas_call(
        _mk_tail(bsz * 64),
        grid=(4,),
        out_shape=jax.ShapeDtypeStruct((bsz * 256, 12), jnp.float32),
        in_specs=[half((bsz * 16, 256)), rep((2, 2, 256)),
                  rep(gamma_3.shape), rep(beta_3.shape),
                  rep((1152, 12)), rep(bias_4.shape)],
        out_specs=half((bsz * 64, 12)),
        compiler_params=par,
    )(acc3, sums3, gamma_3, beta_3, w4p, bias_4)

    out = y4.reshape(bsz, 16, 16, 2, 2, 3)
    out = jnp.transpose(out, (0, 1, 3, 2, 4, 5)).reshape(bsz, 32, 32, 3)
    return jnp.transpose(out, (0, 3, 1, 2))


# speedup vs baseline: 1.0706x; 1.0706x over previous
import jax
import jax.numpy as jnp
from jax.experimental import pallas as pl
from jax.experimental.pallas import tpu as pltpu

_OFFSETS = [(-1, -1), (-1, 0), (-1, 1),
            (0, -1), (0, 0), (0, 1),
            (1, -1), (1, 0), (1, 1)]


def _patches(x):
    b, h, w, c = x.shape
    zrow = jnp.zeros((b, 1, w, c), x.dtype)
    xp = jnp.concatenate([zrow, x, zrow], axis=1)
    zcol = jnp.zeros((b, h + 2, 1, c), x.dtype)
    xp = jnp.concatenate([zcol, xp, zcol], axis=2)
    slabs = [xp[:, 1 + dh:1 + dh + h, 1 + dw:1 + dw + w, :]
             for dh, dw in _OFFSETS]
    cols = jnp.concatenate(slabs, axis=-1)
    return cols.reshape(b * h * w, 9 * c)


def _pixel_shuffle(y, b, h, w, c):
    quads = [y[:, p * c:(p + 1) * c].reshape(b, h, w, 1, c) for p in range(4)]
    row0 = jnp.concatenate([quads[0], quads[1]], axis=3).reshape(b, h, 1, 2 * w, c)
    row1 = jnp.concatenate([quads[2], quads[3]], axis=3).reshape(b, h, 1, 2 * w, c)
    return jnp.concatenate([row0, row1], axis=2).reshape(b, 2 * h, 2 * w, c)


def _col_stats(acc):
    s = jnp.sum(acc, axis=0, keepdims=True)
    q = jnp.sum(acc * acc, axis=0, keepdims=True)
    return jnp.concatenate([s[:, None, :], q[:, None, :]], axis=1)


def _finish_bn(acc, sums, g, t, m_full):
    n = acc.shape[1]
    c = n // 4
    col_sum = sums[0, 0:1, :] + sums[1, 0:1, :]
    col_sq = sums[0, 1:2, :] + sums[1, 1:2, :]
    ch_sum = (col_sum[:, 0:c] + col_sum[:, c:2 * c]
              + col_sum[:, 2 * c:3 * c] + col_sum[:, 3 * c:4 * c])
    ch_sq = (col_sq[:, 0:c] + col_sq[:, c:2 * c]
             + col_sq[:, 2 * c:3 * c] + col_sq[:, 3 * c:4 * c])
    cnt = jnp.float32(m_full * 4)
    mean = ch_sum / cnt
    var = ch_sq / cnt - mean * mean
    inv = jax.lax.rsqrt(var + 1e-5)
    scale = g[:, 0:c] * inv
    shift = t[:, 0:c] - mean * scale
    scale = jnp.concatenate([scale] * 4, axis=1)
    shift = jnp.concatenate([shift] * 4, axis=1)
    acc = acc * scale + shift
    return jnp.where(acc >= 0, acc, 0.05 * acc)


def _fc_d1_kernel(x_ref, w0_ref, b0_ref, w1_ref, acc_ref, sums_ref):
    bh = x_ref.shape[0]
    y0 = jnp.dot(x_ref[...], w0_ref[...],
                 preferred_element_type=jnp.float32) + b0_ref[...]
    y0 = y0.astype(jnp.bfloat16)
    c0 = 512
    cell = [y0[:, q * c0:(q + 1) * c0].reshape(bh, 1, 1, c0) for q in range(4)]
    top = jnp.concatenate([cell[0], cell[1]], axis=2)
    bot = jnp.concatenate([cell[2], cell[3]], axis=2)
    x0 = jnp.concatenate([top, bot], axis=1)
    acc = jnp.dot(_patches(x0), w1_ref[...], preferred_element_type=jnp.float32)
    sums_ref[...] = _col_stats(acc)
    acc_ref[...] = acc.astype(jnp.bfloat16)


def _mk_bn_step(h, w, cin, m_full):
    def body(acc_ref, sums_ref, g_ref, t_ref, w_ref, oacc_ref, osums_ref):
        bh = acc_ref.shape[0] // (h * w)
        acc = _finish_bn(acc_ref[...].astype(jnp.float32), sums_ref[...],
                         g_ref[...], t_ref[...], m_full)
        x = _pixel_shuffle(acc.astype(jnp.bfloat16), bh, h, w, cin)
        nacc = jnp.dot(_patches(x), w_ref[...],
                       preferred_element_type=jnp.float32)
        osums_ref[...] = _col_stats(nacc)
        oacc_ref[...] = nacc.astype(jnp.bfloat16)
    return body


def _finish_d3_kernel(acc_ref, sums_ref, g_ref, t_ref, o_ref):
    bh = acc_ref.shape[0] // 64
    acc = _finish_bn(acc_ref[...].astype(jnp.float32), sums_ref[...],
                     g_ref[...], t_ref[...], 2 * acc_ref.shape[0])
    o_ref[...] = _pixel_shuffle(acc.astype(jnp.bfloat16), bh, 8, 8, 64)


def _head_kernel(x_ref, w_ref, b_ref, o_ref):
    p = _patches(x_ref[...])
    y = jnp.dot(p, w_ref[...], preferred_element_type=jnp.float32)
    o_ref[...] = jnp.tanh(y + b_ref[...])


def kernel(z, tags, wmat_0, bias_0, gamma_0, beta_0,
           wmat_1, bias_1, gamma_1, beta_1,
           wmat_2, bias_2, gamma_2, beta_2,
           wmat_3, bias_3, gamma_3, beta_3,
           wmat_4, bias_4, gamma_4, beta_4):
    bsz = z.shape[0]
    x = jnp.concatenate([z, tags], axis=1).astype(jnp.bfloat16)

    par = pltpu.CompilerParams(dimension_semantics=("parallel",))
    rep = lambda shape: pl.BlockSpec(shape, lambda g: tuple(0 for _ in shape))
    half = lambda shape: pl.BlockSpec(shape, lambda g: (g,) + tuple(
        0 for _ in shape[1:]))

    def stats_shape(n):
        return jax.ShapeDtypeStruct((2, 2, n), jnp.float32)

    acc1, sums1 = pl.pallas_call(
        _fc_d1_kernel,
        grid=(2,),
        out_shape=(jax.ShapeDtypeStruct((bsz * 4, 1024), jnp.bfloat16),
                   stats_shape(1024)),
        in_specs=[half((bsz // 2, x.shape[1])),
                  rep(wmat_0.shape), rep(bias_0.shape), rep(wmat_1.shape)],
        out_specs=(half((bsz * 2, 1024)), half((1, 2, 1024))),
        compiler_params=par,
    )(x, wmat_0, bias_0, wmat_1)

    acc2, sums2 = pl.pallas_call(
        _mk_bn_step(2, 2, 256, bsz * 4),
        grid=(2,),
        out_shape=(jax.ShapeDtypeStruct((bsz * 16, 512), jnp.bfloat16),
                   stats_shape(512)),
        in_specs=[half((bsz * 2, 1024)), rep((2, 2, 1024)),
                  rep(gamma_1.shape), rep(beta_1.shape), rep(wmat_2.shape)],
        out_specs=(half((bsz * 8, 512)), half((1, 2, 512))),
        compiler_params=par,
    )(acc1, sums1, gamma_1, beta_1, wmat_2)

    acc3, sums3 = pl.pallas_call(
        _mk_bn_step(4, 4, 128, bsz * 16),
        grid=(2,),
        out_shape=(jax.ShapeDtypeStruct((bsz * 64, 256), jnp.bfloat16),
                   stats_shape(256)),
        in_specs=[half((bsz * 8, 512)), rep((2, 2, 512)),
                  rep(gamma_2.shape), rep(beta_2.shape), rep(wmat_3.shape)],
        out_specs=(half((bsz * 32, 256)), half((1, 2, 256))),
        compiler_params=par,
    )(acc2, sums2, gamma_2, beta_2, wmat_3)

    x3 = pl.pallas_call(
        _finish_d3_kernel,
        grid=(2,),
        out_shape=jax.ShapeDtypeStruct((bsz, 16, 16, 64), jnp.bfloat16),
        in_specs=[half((bsz * 32, 256)), rep((2, 2, 256)),
                  rep(gamma_3.shape), rep(beta_3.shape)],
        out_specs=half((bsz // 2, 16, 16, 64)),
        compiler_params=par,
    )(acc3, sums3, gamma_3, beta_3)

    y4 = pl.pallas_call(
        _head_kernel,
        grid=(8,),
        out_shape=jax.ShapeDtypeStruct((bsz * 256, 12), jnp.float32),
        in_specs=[half((bsz // 8, 16, 16, 64)),
                  rep(wmat_4.shape), rep(bias_4.shape)],
        out_specs=half((bsz * 32, 12)),
        compiler_params=par,
    )(x3, wmat_4, bias_4)

    out = y4.reshape(bsz, 16, 16, 2, 2, 3)
    out = jnp.transpose(out, (0, 1, 3, 2, 4, 5)).reshape(bsz, 32, 32, 3)
    return jnp.transpose(out, (0, 3, 1, 2))


# speedup vs baseline: 1.1148x; 1.0413x over previous
import jax
import jax.numpy as jnp
from jax.experimental import pallas as pl
from jax.experimental.pallas import tpu as pltpu

_OFFSETS = [(-1, -1), (-1, 0), (-1, 1),
            (0, -1), (0, 0), (0, 1),
            (1, -1), (1, 0), (1, 1)]


def _patches(x):
    b, h, w, c = x.shape
    zrow = jnp.zeros((b, 1, w, c), x.dtype)
    xp = jnp.concatenate([zrow, x, zrow], axis=1)
    zcol = jnp.zeros((b, h + 2, 1, c), x.dtype)
    xp = jnp.concatenate([zcol, xp, zcol], axis=2)
    slabs = [xp[:, 1 + dh:1 + dh + h, 1 + dw:1 + dw + w, :]
             for dh, dw in _OFFSETS]
    cols = jnp.concatenate(slabs, axis=-1)
    return cols.reshape(b * h * w, 9 * c)


def _pixel_shuffle(y, b, h, w, c):
    quads = [y[:, p * c:(p + 1) * c].reshape(b, h, w, 1, c) for p in range(4)]
    row0 = jnp.concatenate([quads[0], quads[1]], axis=3).reshape(b, h, 1, 2 * w, c)
    row1 = jnp.concatenate([quads[2], quads[3]], axis=3).reshape(b, h, 1, 2 * w, c)
    return jnp.concatenate([row0, row1], axis=2).reshape(b, 2 * h, 2 * w, c)


def _col_stats(acc):
    s = jnp.sum(acc, axis=0, keepdims=True)
    q = jnp.sum(acc * acc, axis=0, keepdims=True)
    return jnp.concatenate([s[:, None, :], q[:, None, :]], axis=1)


def _finish_bn(acc, sums, g, t, m_full):
    n = acc.shape[1]
    c = n // 4
    col_sum = sums[0, 0:1, :] + sums[1, 0:1, :]
    col_sq = sums[0, 1:2, :] + sums[1, 1:2, :]
    ch_sum = (col_sum[:, 0:c] + col_sum[:, c:2 * c]
              + col_sum[:, 2 * c:3 * c] + col_sum[:, 3 * c:4 * c])
    ch_sq = (col_sq[:, 0:c] + col_sq[:, c:2 * c]
             + col_sq[:, 2 * c:3 * c] + col_sq[:, 3 * c:4 * c])
    cnt = jnp.float32(m_full * 4)
    mean = ch_sum / cnt
    var = ch_sq / cnt - mean * mean
    inv = jax.lax.rsqrt(var + 1e-5)
    scale = g[:, 0:c] * inv
    shift = t[:, 0:c] - mean * scale
    scale = jnp.concatenate([scale] * 4, axis=1)
    shift = jnp.concatenate([shift] * 4, axis=1)
    acc = acc * scale + shift
    return jnp.where(acc >= 0, acc, 0.05 * acc)


def _fc_d1_kernel(x_ref, w0_ref, b0_ref, w1_ref, acc_ref, sums_ref):
    bh = x_ref.shape[0]
    y0 = jnp.dot(x_ref[...], w0_ref[...],
                 preferred_element_type=jnp.float32) + b0_ref[...]
    y0 = y0.astype(jnp.bfloat16)
    c0 = 512
    cell = [y0[:, q * c0:(q + 1) * c0].reshape(bh, 1, 1, c0) for q in range(4)]
    top = jnp.concatenate([cell[0], cell[1]], axis=2)
    bot = jnp.concatenate([cell[2], cell[3]], axis=2)
    x0 = jnp.concatenate([top, bot], axis=1)
    acc = jnp.dot(_patches(x0), w1_ref[...], preferred_element_type=jnp.float32)
    sums_ref[...] = _col_stats(acc)
    acc_ref[...] = acc.astype(jnp.bfloat16)


def _mk_bn_step(h, w, cin, m_full):
    def body(acc_ref, sums_ref, g_ref, t_ref, w_ref, oacc_ref, osums_ref):
        bh = acc_ref.shape[0] // (h * w)
        acc = _finish_bn(acc_ref[...].astype(jnp.float32), sums_ref[...],
                         g_ref[...], t_ref[...], m_full)
        x = _pixel_shuffle(acc.astype(jnp.bfloat16), bh, h, w, cin)
        nacc = jnp.dot(_patches(x), w_ref[...],
                       preferred_element_type=jnp.float32)
        osums_ref[...] = _col_stats(nacc)
        oacc_ref[...] = nacc.astype(jnp.bfloat16)
    return body


def _tail_kernel(acc_ref, sums_ref, g_ref, t_ref, w_ref, b_ref, o_ref):
    bh = acc_ref.shape[0] // 64
    acc = _finish_bn(acc_ref[...].astype(jnp.float32), sums_ref[...],
                     g_ref[...], t_ref[...], 2 * acc_ref.shape[0])
    x3 = _pixel_shuffle(acc.astype(jnp.bfloat16), bh, 8, 8, 64)
    y = jnp.dot(_patches(x3), w_ref[...], preferred_element_type=jnp.float32)
    o_ref[...] = jnp.tanh(y + b_ref[...])


def kernel(z, tags, wmat_0, bias_0, gamma_0, beta_0,
           wmat_1, bias_1, gamma_1, beta_1,
           wmat_2, bias_2, gamma_2, beta_2,
           wmat_3, bias_3, gamma_3, beta_3,
           wmat_4, bias_4, gamma_4, beta_4):
    bsz = z.shape[0]
    x = jnp.concatenate([z, tags], axis=1).astype(jnp.bfloat16)

    par = pltpu.CompilerParams(dimension_semantics=("parallel",))
    rep = lambda shape: pl.BlockSpec(shape, lambda g: tuple(0 for _ in shape))
    half = lambda shape: pl.BlockSpec(shape, lambda g: (g,) + tuple(
        0 for _ in shape[1:]))

    def stats_shape(n):
        return jax.ShapeDtypeStruct((2, 2, n), jnp.float32)

    acc1, sums1 = pl.pallas_call(
        _fc_d1_kernel,
        grid=(2,),
        out_shape=(jax.ShapeDtypeStruct((bsz * 4, 1024), jnp.bfloat16),
                   stats_shape(1024)),
        in_specs=[half((bsz // 2, x.shape[1])),
                  rep(wmat_0.shape), rep(bias_0.shape), rep(wmat_1.shape)],
        out_specs=(half((bsz * 2, 1024)), half((1, 2, 1024))),
        compiler_params=par,
    )(x, wmat_0, bias_0, wmat_1)

    acc2, sums2 = pl.pallas_call(
        _mk_bn_step(2, 2, 256, bsz * 4),
        grid=(2,),
        out_shape=(jax.ShapeDtypeStruct((bsz * 16, 512), jnp.bfloat16),
                   stats_shape(512)),
        in_specs=[half((bsz * 2, 1024)), rep((2, 2, 1024)),
                  rep(gamma_1.shape), rep(beta_1.shape), rep(wmat_2.shape)],
        out_specs=(half((bsz * 8, 512)), half((1, 2, 512))),
        compiler_params=par,
    )(acc1, sums1, gamma_1, beta_1, wmat_2)

    acc3, sums3 = pl.pallas_call(
        _mk_bn_step(4, 4, 128, bsz * 16),
        grid=(2,),
        out_shape=(jax.ShapeDtypeStruct((bsz * 64, 256), jnp.bfloat16),
                   stats_shape(256)),
        in_specs=[half((bsz * 8, 512)), rep((2, 2, 512)),
                  rep(gamma_2.shape), rep(beta_2.shape), rep(wmat_3.shape)],
        out_specs=(half((bsz * 32, 256)), half((1, 2, 256))),
        compiler_params=par,
    )(acc2, sums2, gamma_2, beta_2, wmat_3)

    y4 = pl.pallas_call(
        _tail_kernel,
        grid=(2,),
        out_shape=jax.ShapeDtypeStruct((bsz * 256, 12), jnp.float32),
        in_specs=[half((bsz * 32, 256)), rep((2, 2, 256)),
                  rep(gamma_3.shape), rep(beta_3.shape),
                  rep(wmat_4.shape), rep(bias_4.shape)],
        out_specs=half((bsz * 128, 12)),
        compiler_params=par,
    )(acc3, sums3, gamma_3, beta_3, wmat_4, bias_4)

    out = y4.reshape(bsz, 16, 16, 2, 2, 3)
    out = jnp.transpose(out, (0, 5, 3, 4, 1, 2))
    out = jax.lax.optimization_barrier(out)
    out = jnp.transpose(out, (0, 1, 4, 2, 5, 3))
    return out.reshape(bsz, 3, 32, 32)
